# baseline (speedup 1.0000x reference)
import functools

import jax
import jax.numpy as jnp
from jax import lax
from jax.experimental import pallas as pl
from jax.experimental.pallas import tpu as pltpu
from jax.experimental.pallas import tpu_sc as plsc

B = 16384
L = 50
D = 64
BAGS_PER_CHUNK = 2
IDX_PER_CHUNK = BAGS_PER_CHUNK * L
CHUNK_PAD = 112
RING = 2


def _make_kernel(n_workers):
    bags_per_w = B // n_workers
    n_chunks = bags_per_w // BAGS_PER_CHUNK
    idx_words = n_chunks * CHUNK_PAD
    mesh = plsc.VectorSubcoreMesh(core_axis_name="c", subcore_axis_name="s")

    @functools.partial(
        pl.kernel,
        mesh=mesh,
        out_type=jax.ShapeDtypeStruct((B * D,), jnp.float32),
        scratch_types=[
            pltpu.VMEM((idx_words,), jnp.int32),
            pltpu.VMEM((idx_words,), jnp.int32),
            pltpu.VMEM((RING, IDX_PER_CHUNK, 2 * D), jnp.float32),
            pltpu.VMEM((bags_per_w * D,), jnp.float32),
        ]
        + [pltpu.SemaphoreType.DMA] * RING,
    )
    def embag(idx_hbm, w_hbm, out_hbm, idx_v, idxr_v, rows_v, out_v, *sems):
        n_cores = lax.axis_size("c")
        wid = lax.axis_index("s") * n_cores + lax.axis_index("c")

        pltpu.sync_copy(idx_hbm.at[pl.ds(wid * idx_words, idx_words)], idx_v)

        def xform_body(i, _):
            v = idx_v[pl.ds(i * 16, 16)]
            idxr_v[pl.ds(i * 16, 16)] = v >> 1
            return ()

        lax.fori_loop(0, idx_words // 16, xform_body, ())

        for b in range(RING):
            pltpu.async_copy(
                w_hbm.at[idxr_v.at[pl.ds(b * CHUNK_PAD, IDX_PER_CHUNK)]],
                rows_v.at[b], sems[b],
            )

        def group_body(p, _):
            for b in range(RING):
                c = p * RING + b
                pltpu.make_async_copy(
                    w_hbm.at[idxr_v.at[pl.ds(c * CHUNK_PAD, IDX_PER_CHUNK)]],
                    rows_v.at[b], sems[b],
                ).wait()
                offv = [
                    (idx_v[pl.ds(c * CHUNK_PAD + j * 16, 16)] & 1) << 6
                    for j in range((IDX_PER_CHUNK + 15) // 16)
                ]
                for bag in range(BAGS_PER_CHUNK):
                    base = bag * L
                    col0 = offv[base // 16][base % 16]
                    acc = [
                        rows_v[b, base, pl.ds(col0 + d * 16, 16)]
                        for d in range(D // 16)
                    ]
                    for r in range(1, L):
                        g = base + r
                        col = offv[g // 16][g % 16]
                        for d in range(D // 16):
                            acc[d] = acc[d] + rows_v[
                                b, g, pl.ds(col + d * 16, 16)
                            ]
                    orow = c * BAGS_PER_CHUNK + bag
                    for d in range(D // 16):
                        out_v[pl.ds(orow * D + d * 16, 16)] = acc[d]

                @pl.when(c + RING < n_chunks)
                def _():
                    pltpu.async_copy(
                        w_hbm.at[
                            idxr_v.at[
                                pl.ds((c + RING) * CHUNK_PAD, IDX_PER_CHUNK)
                            ]
                        ],
                        rows_v.at[b], sems[b],
                    )

            return ()

        lax.fori_loop(0, n_chunks // RING, group_body, ())

        pltpu.sync_copy(
            out_v, out_hbm.at[pl.ds(wid * bags_per_w * D, bags_per_w * D)]
        )

    return embag


@jax.jit
def kernel(indices, W):
    info = plsc.get_sparse_core_info()
    n_workers = info.num_cores * info.num_subcores
    idx = indices.astype(jnp.int32)
    idx2 = jnp.reshape(idx, (B * L // IDX_PER_CHUNK, IDX_PER_CHUNK))
    idxp = jnp.pad(idx2, ((0, 0), (0, CHUNK_PAD - IDX_PER_CHUNK)))
    w2 = jnp.reshape(W, (W.shape[0] // 2, 2 * D))
    out = _make_kernel(n_workers)(jnp.reshape(idxp, (-1,)), w2)
    return jnp.reshape(out, (B, D))


# speedup vs baseline: 1.1365x; 1.1365x over previous
import functools

import jax
import jax.numpy as jnp
from jax import lax
from jax.experimental import pallas as pl
from jax.experimental.pallas import tpu as pltpu
from jax.experimental.pallas import tpu_sc as plsc

B = 16384
L = 50
D = 64
RING = 4


def _make_kernel(n_workers):
    bags_per_w = B // n_workers
    mesh = plsc.VectorSubcoreMesh(core_axis_name="c", subcore_axis_name="s")

    @functools.partial(
        pl.kernel,
        mesh=mesh,
        out_type=jax.ShapeDtypeStruct((B * D,), jnp.float32),
        compiler_params=pltpu.CompilerParams(use_tc_tiling_on_sc=False),
        scratch_types=[
            pltpu.VMEM((bags_per_w, L), jnp.int32),
            pltpu.VMEM((RING, L, D), jnp.float32),
            pltpu.VMEM((bags_per_w * D,), jnp.float32),
        ]
        + [pltpu.SemaphoreType.DMA] * RING,
    )
    def embag(idx_hbm, w_hbm, out_hbm, idx_v, rows_v, out_v, *sems):
        n_cores = lax.axis_size("c")
        wid = lax.axis_index("s") * n_cores + lax.axis_index("c")

        pltpu.sync_copy(idx_hbm.at[pl.ds(wid * bags_per_w, bags_per_w), :],
                        idx_v)

        for b in range(RING):
            pltpu.async_copy(w_hbm.at[idx_v.at[b]], rows_v.at[b], sems[b])

        def group_body(p, _):
            for b in range(RING):
                c = p * RING + b
                pltpu.make_async_copy(
                    w_hbm.at[idx_v.at[c]], rows_v.at[b], sems[b]
                ).wait()
                acc = [rows_v[b, 0, pl.ds(d * 16, 16)] for d in range(D // 16)]
                for r in range(1, L):
                    for d in range(D // 16):
                        acc[d] = acc[d] + rows_v[b, r, pl.ds(d * 16, 16)]
                for d in range(D // 16):
                    out_v[pl.ds(c * D + d * 16, 16)] = acc[d]

                @pl.when(c + RING < bags_per_w)
                def _():
                    pltpu.async_copy(
                        w_hbm.at[idx_v.at[c + RING]], rows_v.at[b], sems[b]
                    )

            return ()

        lax.fori_loop(0, bags_per_w // RING, group_body, ())

        pltpu.sync_copy(
            out_v, out_hbm.at[pl.ds(wid * bags_per_w * D, bags_per_w * D)]
        )

    return embag


@jax.jit
def kernel(indices, W):
    info = plsc.get_sparse_core_info()
    n_workers = info.num_cores * info.num_subcores
    out = _make_kernel(n_workers)(indices.astype(jnp.int32), W)
    return jnp.reshape(out, (B, D))
